# baseline (speedup 1.0000x reference)
import functools

import jax
import jax.numpy as jnp
from jax import lax
from jax.experimental import pallas as pl
from jax.experimental.pallas import tpu as pltpu
from jax.experimental.pallas import tpu_sc as plsc

NUM_LEVELS = 3
DECAY = 0.8
CR0 = 0.5


def _layer_dims_(h):
    dims = []
    cur = h
    cr = CR0
    for _ in range(NUM_LEVELS):
        out = max(int(cur * cr), 1)
        dims.append((cur, out))
        cur = out
        cr *= DECAY
    return dims


def _level_sizes_(total):
    sizes = []
    remaining = total
    for i in range(NUM_LEVELS):
        if i == NUM_LEVELS - 1:
            sizes.append(remaining)
        else:
            sz = int(remaining * (1.0 - DECAY) * (DECAY ** i))
            sizes.append(sz)
            remaining -= sz
    return sizes


_N = 16384
_W = 16
_C = _N // _W
_NB = 1024
_NCH = _C // 16


def _sc_route_body(k0, k1, imp_hbm, lvl_hbm,
                   imp_loc, buck_loc, histred, hist16, scal_loc,
                   mval0, midx0, mval1, midx1,
                   mvalall, midxall, verdfull, rverd0, rverd1,
                   cnt0_loc, cnt1_loc, cntrow,
                   in0_loc, in1_loc, lvl_loc,
                   sh_hist, sh_scal, sh_cnt0, sh_cnt1,
                   sh_val0, sh_idx0, sh_val1, sh_idx1,
                   sh_verd0, sh_verd1):
    cid = lax.axis_index("c")
    sid = lax.axis_index("s")

    @pl.when(cid == 0)
    def _body():
        iota = lax.iota(jnp.int32, 16)
        ones = jnp.ones((16,), jnp.float32)
        zero16 = jnp.zeros((16,), jnp.float32)
        k0f, k1f = float(k0), float(k1)

        def bucket_of(v):
            return jnp.minimum((v * float(_NB)).astype(jnp.int32), _NB - 1)

        pltpu.sync_copy(imp_hbm.at[pl.ds(sid * _C, _C)], imp_loc)

        def zbody(i, _):
            histred[pl.ds(i * 16, 16)] = zero16
            return 0
        lax.fori_loop(0, _NB // 16, zbody, 0)

        def hbody(c, _):
            v = imp_loc[pl.ds(c * 16, 16)]
            b = bucket_of(v)
            buck_loc[pl.ds(c * 16, 16)] = b
            plsc.addupdate_scatter(histred, [b], ones)
            return 0
        lax.fori_loop(0, _NCH, hbody, 0)

        pltpu.sync_copy(histred, sh_hist.at[sid])
        plsc.subcore_barrier()

        @pl.when(sid == 0)
        def _scan():
            pltpu.sync_copy(sh_hist, hist16)

            def sbody(i, carry):
                carryc, b0a, n0a, b1a, n1a = carry
                c = (_NB // 16 - 1) - i
                chunk = jnp.zeros((16,), jnp.float32)
                for w in range(_W):
                    chunk = chunk + hist16[w, pl.ds(c * 16, 16)]
                rc = lax.rev(plsc.cumsum(lax.rev(chunk, (0,))), (0,))
                sufex = carryc + rc - chunk
                bids = (c * 16 + iota).astype(jnp.float32)
                m0 = (sufex < k0f) & (sufex + chunk >= k0f)
                m1 = (sufex < k1f) & (sufex + chunk >= k1f)
                b0a = b0a + jnp.sum(jnp.where(m0, bids, 0.0))
                n0a = n0a + jnp.sum(jnp.where(m0, k0f - sufex, 0.0))
                b1a = b1a + jnp.sum(jnp.where(m1, bids, 0.0))
                n1a = n1a + jnp.sum(jnp.where(m1, k1f - sufex, 0.0))
                return (carryc + jnp.sum(chunk), b0a, n0a, b1a, n1a)

            z = jnp.float32(0.0)
            _c, b0, n0, b1, n1 = lax.fori_loop(0, _NB // 16, sbody,
                                               (z, z, z, z, z))
            scal = (jnp.where(iota == 0, b0, 0.0)
                    + jnp.where(iota == 1, n0, 0.0)
                    + jnp.where(iota == 2, b1, 0.0)
                    + jnp.where(iota == 3, n1, 0.0))
            scal_loc[...] = scal
            pltpu.sync_copy(scal_loc, sh_scal)

        plsc.subcore_barrier()

        pltpu.sync_copy(sh_scal, scal_loc)
        sv = scal_loc[...]

        def pick(j):
            return jnp.sum(jnp.where(iota == j, sv, 0.0))
        b0i = pick(0).astype(jnp.int32)
        need0 = pick(1).astype(jnp.int32)
        b1i = pick(2).astype(jnp.int32)
        need1 = pick(3).astype(jnp.int32)

        def mbody(c, offs):
            off0, off1 = offs
            v = imp_loc[pl.ds(c * 16, 16)]
            b = buck_loc[pl.ds(c * 16, 16)]
            gidx = sid * _C + c * 16 + iota
            m0 = b == b0i
            m1 = b == b1i
            plsc.store_compressed(mval0.at[pl.ds(off0, 16)], v, mask=m0)
            plsc.store_compressed(midx0.at[pl.ds(off0, 16)], gidx, mask=m0)
            plsc.store_compressed(mval1.at[pl.ds(off1, 16)], v, mask=m1)
            plsc.store_compressed(midx1.at[pl.ds(off1, 16)], gidx, mask=m1)
            return (off0 + jnp.sum(m0.astype(jnp.int32)),
                    off1 + jnp.sum(m1.astype(jnp.int32)))

        off0, off1 = lax.fori_loop(0, _NCH, mbody,
                                   (jnp.int32(0), jnp.int32(0)))

        pltpu.sync_copy(mval0.at[pl.ds(0, _C)], sh_val0.at[pl.ds(sid * _C, _C)])
        pltpu.sync_copy(midx0.at[pl.ds(0, _C)], sh_idx0.at[pl.ds(sid * _C, _C)])
        pltpu.sync_copy(mval1.at[pl.ds(0, _C)], sh_val1.at[pl.ds(sid * _C, _C)])
        pltpu.sync_copy(midx1.at[pl.ds(0, _C)], sh_idx1.at[pl.ds(sid * _C, _C)])
        cntrow[...] = jnp.where(iota == 0, off0, 0)
        pltpu.sync_copy(cntrow, sh_cnt0.at[pl.ds(sid * 16, 16)])
        cntrow[...] = jnp.where(iota == 0, off1, 0)
        pltpu.sync_copy(cntrow, sh_cnt1.at[pl.ds(sid * 16, 16)])
        plsc.subcore_barrier()

        @pl.when(sid == 0)
        def _verd():
            pltpu.sync_copy(sh_cnt0, cnt0_loc)
            pltpu.sync_copy(sh_cnt1, cnt1_loc)
            for cntl, needt, shv, shi, shverd in (
                    (cnt0_loc, need0, sh_val0, sh_idx0, sh_verd0),
                    (cnt1_loc, need1, sh_val1, sh_idx1, sh_verd1)):
                pltpu.sync_copy(shv, mvalall)
                pltpu.sync_copy(shi, midxall)

                def cnt_of(w, cntl=cntl):
                    return jnp.sum(jnp.where(iota == 0,
                                             cntl[pl.ds(w * 16, 16)], 0))

                def w1body(w1, _, cnt_of=cnt_of, needt=needt):
                    c1 = cnt_of(w1)

                    def e1body(ec, _):
                        base1 = w1 * _C + ec * 16
                        ov = mvalall[pl.ds(base1, 16)]
                        oi = midxall[pl.ds(base1, 16)]

                        def w2body(w2, rank):
                            c2 = cnt_of(w2)

                            def jbody(j, r):
                                jc = (j // 16) * 16
                                lane = j - jc
                                lm = iota == lane
                                mv = jnp.sum(jnp.where(
                                    lm, mvalall[pl.ds(w2 * _C + jc, 16)], 0.0))
                                mi = jnp.sum(jnp.where(
                                    lm, midxall[pl.ds(w2 * _C + jc, 16)], 0))
                                beats = (mv > ov) | ((mv == ov) & (mi < oi))
                                return r + jnp.where(beats, 1, 0)

                            return lax.fori_loop(0, c2, jbody, rank)

                        rank = lax.fori_loop(0, _W, w2body,
                                             jnp.zeros((16,), jnp.int32))
                        verdict = jnp.where(rank < needt, 1.0, 0.0)
                        verdfull[pl.ds(base1, 16)] = verdict
                        return 0

                    lax.fori_loop(0, (c1 + 15) // 16, e1body, 0)
                    return 0

                lax.fori_loop(0, _W, w1body, 0)
                pltpu.sync_copy(verdfull, shverd)

        plsc.subcore_barrier()

        pltpu.sync_copy(sh_verd0.at[pl.ds(sid * _C, _C)], rverd0)
        pltpu.sync_copy(sh_verd1.at[pl.ds(sid * _C, _C)], rverd1)

        def ibody(c, _):
            b = buck_loc[pl.ds(c * 16, 16)]
            in0_loc[pl.ds(c * 16, 16)] = (b > b0i).astype(jnp.int32)
            in1_loc[pl.ds(c * 16, 16)] = (b > b1i).astype(jnp.int32)
            return 0
        lax.fori_loop(0, _NCH, ibody, 0)

        def obody0(ec, _):
            valid = (ec * 16 + iota) < off0
            pos = midx0[pl.ds(ec * 16, 16)] - sid * _C
            vd = rverd0[pl.ds(ec * 16, 16)].astype(jnp.int32)
            plsc.store_scatter(in0_loc, [pos], vd, mask=valid)
            return 0
        lax.fori_loop(0, (off0 + 15) // 16, obody0, 0)

        def obody1(ec, _):
            valid = (ec * 16 + iota) < off1
            pos = midx1[pl.ds(ec * 16, 16)] - sid * _C
            vd = rverd1[pl.ds(ec * 16, 16)].astype(jnp.int32)
            plsc.store_scatter(in1_loc, [pos], vd, mask=valid)
            return 0
        lax.fori_loop(0, (off1 + 15) // 16, obody1, 0)

        def fbody(c, _):
            i0 = in0_loc[pl.ds(c * 16, 16)]
            i1 = in1_loc[pl.ds(c * 16, 16)]
            lvl_loc[pl.ds(c * 16, 16)] = (2 - i0 - i1).astype(jnp.float32)
            return 0
        lax.fori_loop(0, _NCH, fbody, 0)
        pltpu.sync_copy(lvl_loc, lvl_hbm.at[pl.ds(sid * _C, _C)])


def _sc_compute_levels(imp_flat, k0, k1):
    mesh = plsc.VectorSubcoreMesh(core_axis_name="c", subcore_axis_name="s",
                                  num_cores=2, num_subcores=_W)
    f32, i32 = jnp.float32, jnp.int32
    scratch = [
        pltpu.VMEM((_C,), f32),
        pltpu.VMEM((_C,), i32),
        pltpu.VMEM((_NB,), f32),
        pltpu.VMEM((_W, _NB), f32),
        pltpu.VMEM((16,), f32),
        pltpu.VMEM((_C + 16,), f32),
        pltpu.VMEM((_C + 16,), i32),
        pltpu.VMEM((_C + 16,), f32),
        pltpu.VMEM((_C + 16,), i32),
        pltpu.VMEM((_W * _C,), f32),
        pltpu.VMEM((_W * _C,), i32),
        pltpu.VMEM((_W * _C,), f32),
        pltpu.VMEM((_C,), f32),
        pltpu.VMEM((_C,), f32),
        pltpu.VMEM((_W * 16,), i32),
        pltpu.VMEM((_W * 16,), i32),
        pltpu.VMEM((16,), i32),
        pltpu.VMEM((_C,), i32),
        pltpu.VMEM((_C,), i32),
        pltpu.VMEM((_C,), f32),
        pltpu.VMEM_SHARED((_W, _NB), f32),
        pltpu.VMEM_SHARED((16,), f32),
        pltpu.VMEM_SHARED((_W * 16,), i32),
        pltpu.VMEM_SHARED((_W * 16,), i32),
        pltpu.VMEM_SHARED((_W * _C,), f32),
        pltpu.VMEM_SHARED((_W * _C,), i32),
        pltpu.VMEM_SHARED((_W * _C,), f32),
        pltpu.VMEM_SHARED((_W * _C,), i32),
        pltpu.VMEM_SHARED((_W * _C,), f32),
        pltpu.VMEM_SHARED((_W * _C,), f32),
    ]
    fn = pl.kernel(functools.partial(_sc_route_body, k0, k1),
                   out_type=jax.ShapeDtypeStruct((_N,), f32),
                   mesh=mesh, scratch_types=scratch,
                   compiler_params=pltpu.CompilerParams(
                       needs_layout_passes=False))
    return fn(imp_flat).reshape(_N, 1)


def _dot_t(x, w):
    return lax.dot_general(x, w, (((1,), (1,)), ((), ())),
                           preferred_element_type=jnp.float32)


def _mlp_body(fk_ref, fv_ref, lvl_ref,
              we0_ref, be0_ref, we1_ref, be1_ref,
              wd1_ref, bd1_ref, wd0_ref, bd0_ref,
              ok_ref, ov_ref):
    x = jnp.concatenate([fk_ref[...], fv_ref[...]], axis=0)
    lvl = lvl_ref[...]
    lvl2 = jnp.concatenate([lvl, lvl], axis=0)

    h0 = jnp.maximum(_dot_t(x, we0_ref[...]) + be0_ref[...], 0.0)
    h1 = jnp.maximum(_dot_t(h0, we1_ref[...]) + be1_ref[...], 0.0)
    g1 = jnp.maximum(_dot_t(h1, wd1_ref[...]) + bd1_ref[...], 0.0)
    z = jnp.where(lvl2 == 1.0, h0, g1)
    y = jnp.maximum(_dot_t(z, wd0_ref[...]) + bd0_ref[...], 0.0)
    out = jnp.where(lvl2 == 0.0, x, y)

    t = ok_ref.shape[0]
    ok_ref[...] = out[:t]
    ov_ref[...] = out[t:]


def _run_mlp(fk, fv, lvl, we0, be0, we1p, be1p, wd1p, bd1, wd0, bd0):
    n, h = fk.shape
    t = 256
    d0 = we0.shape[0]
    d1 = we1p.shape[0]

    full = lambda shape: pl.BlockSpec(shape, lambda g: tuple(0 for _ in shape))
    return pl.pallas_call(
        _mlp_body,
        grid=(n // t,),
        in_specs=[
            pl.BlockSpec((t, h), lambda g: (g, 0)),
            pl.BlockSpec((t, h), lambda g: (g, 0)),
            pl.BlockSpec((t, 1), lambda g: (g, 0)),
            full((d0, h)), full((1, d0)),
            full((d1, d0)), full((1, d1)),
            full((d0, d1)), full((1, d0)),
            full((h, d0)), full((1, h)),
        ],
        out_specs=[
            pl.BlockSpec((t, h), lambda g: (g, 0)),
            pl.BlockSpec((t, h), lambda g: (g, 0)),
        ],
        out_shape=[
            jax.ShapeDtypeStruct((n, h), jnp.float32),
            jax.ShapeDtypeStruct((n, h), jnp.float32),
        ],
    )(fk, fv, lvl, we0, be0, we1p, be1p, wd1p, bd1, wd0, bd0)


def kernel(keys, values, importance, We0, be0, We1, be1, We2, be2,
           Wd0, bd0, Wd1, bd1, Wd2, bd2):
    b, s, h = keys.shape
    n = b * s
    sizes = _level_sizes_(n)
    k0 = sizes[0]
    k1 = sizes[0] + sizes[1]

    imp_flat = importance.reshape(n)
    lvl = _sc_compute_levels(imp_flat, k0, k1)

    d1 = We1.shape[0]
    d1p = ((d1 + 127) // 128) * 128
    we1p = jnp.pad(We1, ((0, d1p - d1), (0, 0)))
    be1p = jnp.pad(be1, (0, d1p - d1)).reshape(1, d1p)
    wd1p = jnp.pad(Wd1, ((0, 0), (0, d1p - d1)))

    fk = keys.reshape(n, h)
    fv = values.reshape(n, h)
    ok, ov = _run_mlp(fk, fv, lvl, We0, be0.reshape(1, -1), we1p, be1p,
                      wd1p, bd1.reshape(1, -1), Wd0, bd0.reshape(1, -1))
    return (ok.reshape(b, s, h), ov.reshape(b, s, h))


# speedup vs baseline: 1.1230x; 1.1230x over previous
import functools

import jax
import jax.numpy as jnp
from jax import lax
from jax.experimental import pallas as pl
from jax.experimental.pallas import tpu as pltpu
from jax.experimental.pallas import tpu_sc as plsc

NUM_LEVELS = 3
DECAY = 0.8
CR0 = 0.5


def _layer_dims_(h):
    dims = []
    cur = h
    cr = CR0
    for _ in range(NUM_LEVELS):
        out = max(int(cur * cr), 1)
        dims.append((cur, out))
        cur = out
        cr *= DECAY
    return dims


def _level_sizes_(total):
    sizes = []
    remaining = total
    for i in range(NUM_LEVELS):
        if i == NUM_LEVELS - 1:
            sizes.append(remaining)
        else:
            sz = int(remaining * (1.0 - DECAY) * (DECAY ** i))
            sizes.append(sz)
            remaining -= sz
    return sizes


_N = 16384
_W = 16
_C = _N // _W
_NB = 1024
_NCH = _C // 16


def _sc_route_body(k0, k1, imp_hbm, lvl_hbm,
                   imp_loc, buck_loc, histred, hist16, scal_loc,
                   mval0, midx0, mval1, midx1,
                   mvalall, midxall, verdfull, rverd0, rverd1,
                   cnt0_loc, cnt1_loc, cntrow,
                   in0_loc, in1_loc, lvl_loc,
                   sh_hist, sh_scal, sh_cnt0, sh_cnt1,
                   sh_val0, sh_idx0, sh_val1, sh_idx1,
                   sh_verd0, sh_verd1):
    cid = lax.axis_index("c")
    sid = lax.axis_index("s")

    @pl.when(cid == 0)
    def _body():
        iota = lax.iota(jnp.int32, 16)
        ones = jnp.ones((16,), jnp.float32)
        zero16 = jnp.zeros((16,), jnp.float32)
        k0f, k1f = float(k0), float(k1)

        def bucket_of(v):
            return jnp.minimum((v * float(_NB)).astype(jnp.int32), _NB - 1)

        pltpu.sync_copy(imp_hbm.at[pl.ds(sid * _C, _C)], imp_loc)

        def zbody(i, _):
            histred[pl.ds(i * 16, 16)] = zero16
            return 0
        lax.fori_loop(0, _NB // 16, zbody, 0)

        def hbody(c, _):
            v = imp_loc[pl.ds(c * 16, 16)]
            b = bucket_of(v)
            buck_loc[pl.ds(c * 16, 16)] = b
            plsc.addupdate_scatter(histred, [b], ones)
            return 0
        lax.fori_loop(0, _NCH, hbody, 0)

        pltpu.sync_copy(histred, sh_hist.at[sid])
        plsc.subcore_barrier()

        @pl.when(sid == 0)
        def _scan():
            pltpu.sync_copy(sh_hist, hist16)

            def sbody(i, carry):
                carryc, b0a, n0a, b1a, n1a = carry
                c = (_NB // 16 - 1) - i
                chunk = jnp.zeros((16,), jnp.float32)
                for w in range(_W):
                    chunk = chunk + hist16[w, pl.ds(c * 16, 16)]
                rc = lax.rev(plsc.cumsum(lax.rev(chunk, (0,))), (0,))
                sufex = carryc + rc - chunk
                bids = (c * 16 + iota).astype(jnp.float32)
                m0 = (sufex < k0f) & (sufex + chunk >= k0f)
                m1 = (sufex < k1f) & (sufex + chunk >= k1f)
                b0a = b0a + jnp.sum(jnp.where(m0, bids, 0.0))
                n0a = n0a + jnp.sum(jnp.where(m0, k0f - sufex, 0.0))
                b1a = b1a + jnp.sum(jnp.where(m1, bids, 0.0))
                n1a = n1a + jnp.sum(jnp.where(m1, k1f - sufex, 0.0))
                return (carryc + jnp.sum(chunk), b0a, n0a, b1a, n1a)

            z = jnp.float32(0.0)
            _c, b0, n0, b1, n1 = lax.fori_loop(0, _NB // 16, sbody,
                                               (z, z, z, z, z))
            scal = (jnp.where(iota == 0, b0, 0.0)
                    + jnp.where(iota == 1, n0, 0.0)
                    + jnp.where(iota == 2, b1, 0.0)
                    + jnp.where(iota == 3, n1, 0.0))
            scal_loc[...] = scal
            pltpu.sync_copy(scal_loc, sh_scal)

        plsc.subcore_barrier()

        pltpu.sync_copy(sh_scal, scal_loc)
        sv = scal_loc[...]

        def pick(j):
            return jnp.sum(jnp.where(iota == j, sv, 0.0))
        b0i = pick(0).astype(jnp.int32)
        need0 = pick(1).astype(jnp.int32)
        b1i = pick(2).astype(jnp.int32)
        need1 = pick(3).astype(jnp.int32)

        def mbody(c, offs):
            off0, off1 = offs
            v = imp_loc[pl.ds(c * 16, 16)]
            b = buck_loc[pl.ds(c * 16, 16)]
            gidx = sid * _C + c * 16 + iota
            m0 = b == b0i
            m1 = b == b1i
            plsc.store_compressed(mval0.at[pl.ds(off0, 16)], v, mask=m0)
            plsc.store_compressed(midx0.at[pl.ds(off0, 16)], gidx, mask=m0)
            plsc.store_compressed(mval1.at[pl.ds(off1, 16)], v, mask=m1)
            plsc.store_compressed(midx1.at[pl.ds(off1, 16)], gidx, mask=m1)
            return (off0 + jnp.sum(m0.astype(jnp.int32)),
                    off1 + jnp.sum(m1.astype(jnp.int32)))

        off0, off1 = lax.fori_loop(0, _NCH, mbody,
                                   (jnp.int32(0), jnp.int32(0)))

        pltpu.sync_copy(mval0.at[pl.ds(0, _C)], sh_val0.at[pl.ds(sid * _C, _C)])
        pltpu.sync_copy(midx0.at[pl.ds(0, _C)], sh_idx0.at[pl.ds(sid * _C, _C)])
        pltpu.sync_copy(mval1.at[pl.ds(0, _C)], sh_val1.at[pl.ds(sid * _C, _C)])
        pltpu.sync_copy(midx1.at[pl.ds(0, _C)], sh_idx1.at[pl.ds(sid * _C, _C)])
        cntrow[...] = jnp.where(iota == 0, off0, 0)
        pltpu.sync_copy(cntrow, sh_cnt0.at[pl.ds(sid * 16, 16)])
        cntrow[...] = jnp.where(iota == 0, off1, 0)
        pltpu.sync_copy(cntrow, sh_cnt1.at[pl.ds(sid * 16, 16)])
        plsc.subcore_barrier()

        @pl.when(sid == 0)
        def _verd():
            pltpu.sync_copy(sh_cnt0, cnt0_loc)
            pltpu.sync_copy(sh_cnt1, cnt1_loc)
            for cntl, needt, shv, shi, shverd in (
                    (cnt0_loc, need0, sh_val0, sh_idx0, sh_verd0),
                    (cnt1_loc, need1, sh_val1, sh_idx1, sh_verd1)):
                pltpu.sync_copy(shv, mvalall)
                pltpu.sync_copy(shi, midxall)

                def cnt_of(w, cntl=cntl):
                    return jnp.sum(jnp.where(iota == 0,
                                             cntl[pl.ds(w * 16, 16)], 0))

                def w1body(w1, _, cnt_of=cnt_of, needt=needt):
                    c1 = cnt_of(w1)

                    def e1body(ec, _):
                        base1 = w1 * _C + ec * 16
                        ov = mvalall[pl.ds(base1, 16)]
                        oi = midxall[pl.ds(base1, 16)]

                        def w2body(w2, rank):
                            c2 = cnt_of(w2)

                            def jbody(j, r):
                                jc = (j // 16) * 16
                                lane = j - jc
                                lm = iota == lane
                                mv = jnp.sum(jnp.where(
                                    lm, mvalall[pl.ds(w2 * _C + jc, 16)], 0.0))
                                mi = jnp.sum(jnp.where(
                                    lm, midxall[pl.ds(w2 * _C + jc, 16)], 0))
                                beats = (mv > ov) | ((mv == ov) & (mi < oi))
                                return r + jnp.where(beats, 1, 0)

                            return lax.fori_loop(0, c2, jbody, rank)

                        rank = lax.fori_loop(0, _W, w2body,
                                             jnp.zeros((16,), jnp.int32))
                        verdict = jnp.where(rank < needt, 1.0, 0.0)
                        verdfull[pl.ds(base1, 16)] = verdict
                        return 0

                    lax.fori_loop(0, (c1 + 15) // 16, e1body, 0)
                    return 0

                lax.fori_loop(0, _W, w1body, 0)
                pltpu.sync_copy(verdfull, shverd)

        plsc.subcore_barrier()

        pltpu.sync_copy(sh_verd0.at[pl.ds(sid * _C, _C)], rverd0)
        pltpu.sync_copy(sh_verd1.at[pl.ds(sid * _C, _C)], rverd1)

        def ibody(c, _):
            b = buck_loc[pl.ds(c * 16, 16)]
            in0_loc[pl.ds(c * 16, 16)] = (b > b0i).astype(jnp.int32)
            in1_loc[pl.ds(c * 16, 16)] = (b > b1i).astype(jnp.int32)
            return 0
        lax.fori_loop(0, _NCH, ibody, 0)

        def obody0(ec, _):
            valid = (ec * 16 + iota) < off0
            pos = midx0[pl.ds(ec * 16, 16)] - sid * _C
            vd = rverd0[pl.ds(ec * 16, 16)].astype(jnp.int32)
            plsc.store_scatter(in0_loc, [pos], vd, mask=valid)
            return 0
        lax.fori_loop(0, (off0 + 15) // 16, obody0, 0)

        def obody1(ec, _):
            valid = (ec * 16 + iota) < off1
            pos = midx1[pl.ds(ec * 16, 16)] - sid * _C
            vd = rverd1[pl.ds(ec * 16, 16)].astype(jnp.int32)
            plsc.store_scatter(in1_loc, [pos], vd, mask=valid)
            return 0
        lax.fori_loop(0, (off1 + 15) // 16, obody1, 0)

        def fbody(c, _):
            i0 = in0_loc[pl.ds(c * 16, 16)]
            i1 = in1_loc[pl.ds(c * 16, 16)]
            lvl_loc[pl.ds(c * 16, 16)] = (2 - i0 - i1).astype(jnp.float32)
            return 0
        lax.fori_loop(0, _NCH, fbody, 0)
        pltpu.sync_copy(lvl_loc, lvl_hbm.at[pl.ds(sid * _C, _C)])


def _sc_compute_levels(imp_flat, k0, k1):
    mesh = plsc.VectorSubcoreMesh(core_axis_name="c", subcore_axis_name="s",
                                  num_cores=2, num_subcores=_W)
    f32, i32 = jnp.float32, jnp.int32
    scratch = [
        pltpu.VMEM((_C,), f32),
        pltpu.VMEM((_C,), i32),
        pltpu.VMEM((_NB,), f32),
        pltpu.VMEM((_W, _NB), f32),
        pltpu.VMEM((16,), f32),
        pltpu.VMEM((_C + 16,), f32),
        pltpu.VMEM((_C + 16,), i32),
        pltpu.VMEM((_C + 16,), f32),
        pltpu.VMEM((_C + 16,), i32),
        pltpu.VMEM((_W * _C,), f32),
        pltpu.VMEM((_W * _C,), i32),
        pltpu.VMEM((_W * _C,), f32),
        pltpu.VMEM((_C,), f32),
        pltpu.VMEM((_C,), f32),
        pltpu.VMEM((_W * 16,), i32),
        pltpu.VMEM((_W * 16,), i32),
        pltpu.VMEM((16,), i32),
        pltpu.VMEM((_C,), i32),
        pltpu.VMEM((_C,), i32),
        pltpu.VMEM((_C,), f32),
        pltpu.VMEM_SHARED((_W, _NB), f32),
        pltpu.VMEM_SHARED((16,), f32),
        pltpu.VMEM_SHARED((_W * 16,), i32),
        pltpu.VMEM_SHARED((_W * 16,), i32),
        pltpu.VMEM_SHARED((_W * _C,), f32),
        pltpu.VMEM_SHARED((_W * _C,), i32),
        pltpu.VMEM_SHARED((_W * _C,), f32),
        pltpu.VMEM_SHARED((_W * _C,), i32),
        pltpu.VMEM_SHARED((_W * _C,), f32),
        pltpu.VMEM_SHARED((_W * _C,), f32),
    ]
    fn = pl.kernel(functools.partial(_sc_route_body, k0, k1),
                   out_type=jax.ShapeDtypeStruct((_N,), f32),
                   mesh=mesh, scratch_types=scratch,
                   compiler_params=pltpu.CompilerParams(
                       needs_layout_passes=False))
    return fn(imp_flat).reshape(_N, 1)


def _dot_t(x, w):
    return lax.dot_general(x, w, (((1,), (1,)), ((), ())),
                           preferred_element_type=jnp.float32)


def _mlp_body(fk_ref, fv_ref, lvl_ref,
              we0_ref, be0_ref, we1_ref, be1_ref,
              wd1_ref, bd1_ref, wd0_ref, bd0_ref,
              ok_ref, ov_ref):
    x = jnp.concatenate([fk_ref[...], fv_ref[...]], axis=0)
    lvl = lvl_ref[...]
    lvl2 = jnp.concatenate([lvl, lvl], axis=0)

    h0 = jnp.maximum(_dot_t(x, we0_ref[...]) + be0_ref[...], 0.0)
    h1 = jnp.maximum(_dot_t(h0, we1_ref[...]) + be1_ref[...], 0.0)
    g1 = jnp.maximum(_dot_t(h1, wd1_ref[...]) + bd1_ref[...], 0.0)
    z = jnp.where(lvl2 == 1.0, h0, g1)
    y = jnp.maximum(_dot_t(z, wd0_ref[...]) + bd0_ref[...], 0.0)
    out = jnp.where(lvl2 == 0.0, x, y)

    t = ok_ref.shape[0]
    ok_ref[...] = out[:t]
    ov_ref[...] = out[t:]


def _run_mlp(fk, fv, lvl, we0, be0, we1p, be1p, wd1p, bd1, wd0, bd0):
    n, h = fk.shape
    t = 512
    d0 = we0.shape[0]
    d1 = we1p.shape[0]

    full = lambda shape: pl.BlockSpec(shape, lambda g: tuple(0 for _ in shape))
    return pl.pallas_call(
        _mlp_body,
        grid=(n // t,),
        in_specs=[
            pl.BlockSpec((t, h), lambda g: (g, 0)),
            pl.BlockSpec((t, h), lambda g: (g, 0)),
            pl.BlockSpec((t, 1), lambda g: (g, 0)),
            full((d0, h)), full((1, d0)),
            full((d1, d0)), full((1, d1)),
            full((d0, d1)), full((1, d0)),
            full((h, d0)), full((1, h)),
        ],
        out_specs=[
            pl.BlockSpec((t, h), lambda g: (g, 0)),
            pl.BlockSpec((t, h), lambda g: (g, 0)),
        ],
        out_shape=[
            jax.ShapeDtypeStruct((n, h), jnp.float32),
            jax.ShapeDtypeStruct((n, h), jnp.float32),
        ],
    )(fk, fv, lvl, we0, be0, we1p, be1p, wd1p, bd1, wd0, bd0)


def kernel(keys, values, importance, We0, be0, We1, be1, We2, be2,
           Wd0, bd0, Wd1, bd1, Wd2, bd2):
    b, s, h = keys.shape
    n = b * s
    sizes = _level_sizes_(n)
    k0 = sizes[0]
    k1 = sizes[0] + sizes[1]

    imp_flat = importance.reshape(n)
    lvl = _sc_compute_levels(imp_flat, k0, k1)

    d1 = We1.shape[0]
    d1p = ((d1 + 127) // 128) * 128
    we1p = jnp.pad(We1, ((0, d1p - d1), (0, 0)))
    be1p = jnp.pad(be1, (0, d1p - d1)).reshape(1, d1p)
    wd1p = jnp.pad(Wd1, ((0, 0), (0, d1p - d1)))

    fk = keys.reshape(n, h)
    fv = values.reshape(n, h)
    ok, ov = _run_mlp(fk, fv, lvl, We0, be0.reshape(1, -1), we1p, be1p,
                      wd1p, bd1.reshape(1, -1), Wd0, bd0.reshape(1, -1))
    return (ok.reshape(b, s, h), ov.reshape(b, s, h))


# speedup vs baseline: 1.1568x; 1.0301x over previous
import functools

import jax
import jax.numpy as jnp
from jax import lax
from jax.experimental import pallas as pl
from jax.experimental.pallas import tpu as pltpu
from jax.experimental.pallas import tpu_sc as plsc

NUM_LEVELS = 3
DECAY = 0.8
CR0 = 0.5


def _layer_dims_(h):
    dims = []
    cur = h
    cr = CR0
    for _ in range(NUM_LEVELS):
        out = max(int(cur * cr), 1)
        dims.append((cur, out))
        cur = out
        cr *= DECAY
    return dims


def _level_sizes_(total):
    sizes = []
    remaining = total
    for i in range(NUM_LEVELS):
        if i == NUM_LEVELS - 1:
            sizes.append(remaining)
        else:
            sz = int(remaining * (1.0 - DECAY) * (DECAY ** i))
            sizes.append(sz)
            remaining -= sz
    return sizes


_N = 16384
_W = 16
_C = _N // _W
_NB = 1024
_NCH = _C // 16


def _sc_route_body(k0, k1, imp_hbm, lvl_hbm,
                   imp_loc, buck_loc, histred, hist16, scal_loc,
                   mval0, midx0, mval1, midx1,
                   mvalall, midxall, verdfull, rverd0, rverd1,
                   cnt0_loc, cnt1_loc, cntrow,
                   in0_loc, in1_loc, lvl_loc,
                   sh_hist, sh_scal, sh_cnt0, sh_cnt1,
                   sh_val0, sh_idx0, sh_val1, sh_idx1,
                   sh_verd0, sh_verd1):
    cid = lax.axis_index("c")
    sid = lax.axis_index("s")

    @pl.when(cid == 0)
    def _body():
        iota = lax.iota(jnp.int32, 16)
        ones = jnp.ones((16,), jnp.float32)
        zero16 = jnp.zeros((16,), jnp.float32)
        k0f, k1f = float(k0), float(k1)

        def bucket_of(v):
            return jnp.minimum((v * float(_NB)).astype(jnp.int32), _NB - 1)

        pltpu.sync_copy(imp_hbm.at[pl.ds(sid * _C, _C)], imp_loc)

        def zbody(i, _):
            histred[pl.ds(i * 16, 16)] = zero16
            return 0
        lax.fori_loop(0, _NB // 16, zbody, 0)

        def hbody(c, _):
            v = imp_loc[pl.ds(c * 16, 16)]
            b = bucket_of(v)
            buck_loc[pl.ds(c * 16, 16)] = b
            plsc.addupdate_scatter(histred, [b], ones)
            return 0
        lax.fori_loop(0, _NCH, hbody, 0)

        pltpu.sync_copy(histred, sh_hist.at[sid])
        plsc.subcore_barrier()

        @pl.when(sid == 0)
        def _scan():
            pltpu.sync_copy(sh_hist, hist16)

            def sbody(i, carry):
                carryc, b0a, n0a, b1a, n1a = carry
                c = (_NB // 16 - 1) - i
                chunk = jnp.zeros((16,), jnp.float32)
                for w in range(_W):
                    chunk = chunk + hist16[w, pl.ds(c * 16, 16)]
                rc = lax.rev(plsc.cumsum(lax.rev(chunk, (0,))), (0,))
                sufex = carryc + rc - chunk
                bids = (c * 16 + iota).astype(jnp.float32)
                m0 = (sufex < k0f) & (sufex + chunk >= k0f)
                m1 = (sufex < k1f) & (sufex + chunk >= k1f)
                b0a = b0a + jnp.sum(jnp.where(m0, bids, 0.0))
                n0a = n0a + jnp.sum(jnp.where(m0, k0f - sufex, 0.0))
                b1a = b1a + jnp.sum(jnp.where(m1, bids, 0.0))
                n1a = n1a + jnp.sum(jnp.where(m1, k1f - sufex, 0.0))
                return (carryc + jnp.sum(chunk), b0a, n0a, b1a, n1a)

            z = jnp.float32(0.0)
            _c, b0, n0, b1, n1 = lax.fori_loop(0, _NB // 16, sbody,
                                               (z, z, z, z, z))
            scal = (jnp.where(iota == 0, b0, 0.0)
                    + jnp.where(iota == 1, n0, 0.0)
                    + jnp.where(iota == 2, b1, 0.0)
                    + jnp.where(iota == 3, n1, 0.0))
            scal_loc[...] = scal
            pltpu.sync_copy(scal_loc, sh_scal)

        plsc.subcore_barrier()

        pltpu.sync_copy(sh_scal, scal_loc)
        sv = scal_loc[...]

        def pick(j):
            return jnp.sum(jnp.where(iota == j, sv, 0.0))
        b0i = pick(0).astype(jnp.int32)
        need0 = pick(1).astype(jnp.int32)
        b1i = pick(2).astype(jnp.int32)
        need1 = pick(3).astype(jnp.int32)

        def mbody(c, offs):
            off0, off1 = offs
            v = imp_loc[pl.ds(c * 16, 16)]
            b = buck_loc[pl.ds(c * 16, 16)]
            gidx = sid * _C + c * 16 + iota
            m0 = b == b0i
            m1 = b == b1i
            plsc.store_compressed(mval0.at[pl.ds(off0, 16)], v, mask=m0)
            plsc.store_compressed(midx0.at[pl.ds(off0, 16)], gidx, mask=m0)
            plsc.store_compressed(mval1.at[pl.ds(off1, 16)], v, mask=m1)
            plsc.store_compressed(midx1.at[pl.ds(off1, 16)], gidx, mask=m1)
            return (off0 + jnp.sum(m0.astype(jnp.int32)),
                    off1 + jnp.sum(m1.astype(jnp.int32)))

        off0, off1 = lax.fori_loop(0, _NCH, mbody,
                                   (jnp.int32(0), jnp.int32(0)))

        pltpu.sync_copy(mval0.at[pl.ds(0, _C)], sh_val0.at[pl.ds(sid * _C, _C)])
        pltpu.sync_copy(midx0.at[pl.ds(0, _C)], sh_idx0.at[pl.ds(sid * _C, _C)])
        pltpu.sync_copy(mval1.at[pl.ds(0, _C)], sh_val1.at[pl.ds(sid * _C, _C)])
        pltpu.sync_copy(midx1.at[pl.ds(0, _C)], sh_idx1.at[pl.ds(sid * _C, _C)])
        cntrow[...] = jnp.where(iota == 0, off0, 0)
        pltpu.sync_copy(cntrow, sh_cnt0.at[pl.ds(sid * 16, 16)])
        cntrow[...] = jnp.where(iota == 0, off1, 0)
        pltpu.sync_copy(cntrow, sh_cnt1.at[pl.ds(sid * 16, 16)])
        plsc.subcore_barrier()

        @pl.when(sid == 0)
        def _verd():
            pltpu.sync_copy(sh_cnt0, cnt0_loc)
            pltpu.sync_copy(sh_cnt1, cnt1_loc)
            for cntl, needt, shv, shi, shverd in (
                    (cnt0_loc, need0, sh_val0, sh_idx0, sh_verd0),
                    (cnt1_loc, need1, sh_val1, sh_idx1, sh_verd1)):
                pltpu.sync_copy(shv, mvalall)
                pltpu.sync_copy(shi, midxall)

                def cnt_of(w, cntl=cntl):
                    return jnp.sum(jnp.where(iota == 0,
                                             cntl[pl.ds(w * 16, 16)], 0))

                def w1body(w1, _, cnt_of=cnt_of, needt=needt):
                    c1 = cnt_of(w1)

                    def e1body(ec, _):
                        base1 = w1 * _C + ec * 16
                        ov = mvalall[pl.ds(base1, 16)]
                        oi = midxall[pl.ds(base1, 16)]

                        def w2body(w2, rank):
                            c2 = cnt_of(w2)

                            def jbody(j, r):
                                jc = (j // 16) * 16
                                lane = j - jc
                                lm = iota == lane
                                mv = jnp.sum(jnp.where(
                                    lm, mvalall[pl.ds(w2 * _C + jc, 16)], 0.0))
                                mi = jnp.sum(jnp.where(
                                    lm, midxall[pl.ds(w2 * _C + jc, 16)], 0))
                                beats = (mv > ov) | ((mv == ov) & (mi < oi))
                                return r + jnp.where(beats, 1, 0)

                            return lax.fori_loop(0, c2, jbody, rank)

                        rank = lax.fori_loop(0, _W, w2body,
                                             jnp.zeros((16,), jnp.int32))
                        verdict = jnp.where(rank < needt, 1.0, 0.0)
                        verdfull[pl.ds(base1, 16)] = verdict
                        return 0

                    lax.fori_loop(0, (c1 + 15) // 16, e1body, 0)
                    return 0

                lax.fori_loop(0, _W, w1body, 0)
                pltpu.sync_copy(verdfull, shverd)

        plsc.subcore_barrier()

        pltpu.sync_copy(sh_verd0.at[pl.ds(sid * _C, _C)], rverd0)
        pltpu.sync_copy(sh_verd1.at[pl.ds(sid * _C, _C)], rverd1)

        def ibody(c, _):
            b = buck_loc[pl.ds(c * 16, 16)]
            in0_loc[pl.ds(c * 16, 16)] = (b > b0i).astype(jnp.int32)
            in1_loc[pl.ds(c * 16, 16)] = (b > b1i).astype(jnp.int32)
            return 0
        lax.fori_loop(0, _NCH, ibody, 0)

        def obody0(ec, _):
            valid = (ec * 16 + iota) < off0
            pos = midx0[pl.ds(ec * 16, 16)] - sid * _C
            vd = rverd0[pl.ds(ec * 16, 16)].astype(jnp.int32)
            plsc.store_scatter(in0_loc, [pos], vd, mask=valid)
            return 0
        lax.fori_loop(0, (off0 + 15) // 16, obody0, 0)

        def obody1(ec, _):
            valid = (ec * 16 + iota) < off1
            pos = midx1[pl.ds(ec * 16, 16)] - sid * _C
            vd = rverd1[pl.ds(ec * 16, 16)].astype(jnp.int32)
            plsc.store_scatter(in1_loc, [pos], vd, mask=valid)
            return 0
        lax.fori_loop(0, (off1 + 15) // 16, obody1, 0)

        def fbody(c, _):
            i0 = in0_loc[pl.ds(c * 16, 16)]
            i1 = in1_loc[pl.ds(c * 16, 16)]
            lvl_loc[pl.ds(c * 16, 16)] = (2 - i0 - i1).astype(jnp.float32)
            return 0
        lax.fori_loop(0, _NCH, fbody, 0)
        pltpu.sync_copy(lvl_loc, lvl_hbm.at[pl.ds(sid * _C, _C)])


def _sc_compute_levels(imp_flat, k0, k1):
    mesh = plsc.VectorSubcoreMesh(core_axis_name="c", subcore_axis_name="s",
                                  num_cores=2, num_subcores=_W)
    f32, i32 = jnp.float32, jnp.int32
    scratch = [
        pltpu.VMEM((_C,), f32),
        pltpu.VMEM((_C,), i32),
        pltpu.VMEM((_NB,), f32),
        pltpu.VMEM((_W, _NB), f32),
        pltpu.VMEM((16,), f32),
        pltpu.VMEM((_C + 16,), f32),
        pltpu.VMEM((_C + 16,), i32),
        pltpu.VMEM((_C + 16,), f32),
        pltpu.VMEM((_C + 16,), i32),
        pltpu.VMEM((_W * _C,), f32),
        pltpu.VMEM((_W * _C,), i32),
        pltpu.VMEM((_W * _C,), f32),
        pltpu.VMEM((_C,), f32),
        pltpu.VMEM((_C,), f32),
        pltpu.VMEM((_W * 16,), i32),
        pltpu.VMEM((_W * 16,), i32),
        pltpu.VMEM((16,), i32),
        pltpu.VMEM((_C,), i32),
        pltpu.VMEM((_C,), i32),
        pltpu.VMEM((_C,), f32),
        pltpu.VMEM_SHARED((_W, _NB), f32),
        pltpu.VMEM_SHARED((16,), f32),
        pltpu.VMEM_SHARED((_W * 16,), i32),
        pltpu.VMEM_SHARED((_W * 16,), i32),
        pltpu.VMEM_SHARED((_W * _C,), f32),
        pltpu.VMEM_SHARED((_W * _C,), i32),
        pltpu.VMEM_SHARED((_W * _C,), f32),
        pltpu.VMEM_SHARED((_W * _C,), i32),
        pltpu.VMEM_SHARED((_W * _C,), f32),
        pltpu.VMEM_SHARED((_W * _C,), f32),
    ]
    fn = pl.kernel(functools.partial(_sc_route_body, k0, k1),
                   out_type=jax.ShapeDtypeStruct((_N,), f32),
                   mesh=mesh, scratch_types=scratch,
                   compiler_params=pltpu.CompilerParams(
                       needs_layout_passes=False))
    return fn(imp_flat).reshape(_N, 1)


def _dot_t(x, w):
    return lax.dot_general(x, w, (((1,), (1,)), ((), ())),
                           preferred_element_type=jnp.float32)


def _mlp_body(fk_ref, fv_ref, lvl_ref,
              we0_ref, be0_ref, we1_ref, be1_ref,
              wd1_ref, bd1_ref, wd0_ref, bd0_ref,
              ok_ref, ov_ref):
    x = jnp.concatenate([fk_ref[...], fv_ref[...]], axis=0)
    lvl = lvl_ref[...]
    lvl2 = jnp.concatenate([lvl, lvl], axis=0)

    h0 = jnp.maximum(_dot_t(x, we0_ref[...]) + be0_ref[...], 0.0)
    h1 = jnp.maximum(_dot_t(h0, we1_ref[...]) + be1_ref[...], 0.0)
    g1 = jnp.maximum(_dot_t(h1, wd1_ref[...]) + bd1_ref[...], 0.0)
    z = jnp.where(lvl2 == 1.0, h0, g1)
    y = jnp.maximum(_dot_t(z, wd0_ref[...]) + bd0_ref[...], 0.0)
    out = jnp.where(lvl2 == 0.0, x, y)

    t = ok_ref.shape[0]
    ok_ref[...] = out[:t]
    ov_ref[...] = out[t:]


def _run_mlp(fk, fv, lvl, we0, be0, we1p, be1p, wd1p, bd1, wd0, bd0):
    n, h = fk.shape
    t = 1024
    d0 = we0.shape[0]
    d1 = we1p.shape[0]

    full = lambda shape: pl.BlockSpec(shape, lambda g: tuple(0 for _ in shape))
    return pl.pallas_call(
        _mlp_body,
        grid=(n // t,),
        in_specs=[
            pl.BlockSpec((t, h), lambda g: (g, 0)),
            pl.BlockSpec((t, h), lambda g: (g, 0)),
            pl.BlockSpec((t, 1), lambda g: (g, 0)),
            full((d0, h)), full((1, d0)),
            full((d1, d0)), full((1, d1)),
            full((d0, d1)), full((1, d0)),
            full((h, d0)), full((1, h)),
        ],
        out_specs=[
            pl.BlockSpec((t, h), lambda g: (g, 0)),
            pl.BlockSpec((t, h), lambda g: (g, 0)),
        ],
        out_shape=[
            jax.ShapeDtypeStruct((n, h), jnp.float32),
            jax.ShapeDtypeStruct((n, h), jnp.float32),
        ],
    )(fk, fv, lvl, we0, be0, we1p, be1p, wd1p, bd1, wd0, bd0)


def kernel(keys, values, importance, We0, be0, We1, be1, We2, be2,
           Wd0, bd0, Wd1, bd1, Wd2, bd2):
    b, s, h = keys.shape
    n = b * s
    sizes = _level_sizes_(n)
    k0 = sizes[0]
    k1 = sizes[0] + sizes[1]

    imp_flat = importance.reshape(n)
    lvl = _sc_compute_levels(imp_flat, k0, k1)

    d1 = We1.shape[0]
    d1p = ((d1 + 127) // 128) * 128
    we1p = jnp.pad(We1, ((0, d1p - d1), (0, 0)))
    be1p = jnp.pad(be1, (0, d1p - d1)).reshape(1, d1p)
    wd1p = jnp.pad(Wd1, ((0, 0), (0, d1p - d1)))

    fk = keys.reshape(n, h)
    fv = values.reshape(n, h)
    ok, ov = _run_mlp(fk, fv, lvl, We0, be0.reshape(1, -1), we1p, be1p,
                      wd1p, bd1.reshape(1, -1), Wd0, bd0.reshape(1, -1))
    return (ok.reshape(b, s, h), ov.reshape(b, s, h))
